# baseline (speedup 1.0000x reference)
import functools

import jax
import jax.numpy as jnp
from jax import lax
from jax.experimental import pallas as pl
from jax.experimental.pallas import tpu as pltpu
from jax.experimental.pallas import tpu_sc as plsc

NUM_FIELDS = 26
VOCAB = 100000
D = 16
B = 16384
OUT_W = NUM_FIELDS * D
GRP = 128 // D
GROUP_SIZES = (8, 8, 8, 2)

NC = 2
NS = 16
NW = NC * NS
BPW = B // NW
R = 128
NCHUNK = BPW // R
L = 16

TBLK = 5120
NSTEP = -(-VOCAB // TBLK)


def _make_pack(nf):
    def body(*refs):
        t_refs = refs[:nf]
        o_ref = refs[nf]
        parts = [t_refs[j][...] for j in range(nf)]
        if nf < GRP:
            parts += [jnp.zeros_like(parts[0])] * (GRP - nf)
        xk = jnp.concatenate(parts, axis=0)
        o_ref[...] = jnp.transpose(xk, (1, 0))

    return pl.pallas_call(
        body,
        grid=(NSTEP,),
        in_specs=[pl.BlockSpec((D, TBLK), lambda i: (0, i))] * nf,
        out_specs=pl.BlockSpec((TBLK, 128), lambda i: (i, 0)),
        out_shape=jax.ShapeDtypeStruct((VOCAB, 128), jnp.float32),
    )


_pack8 = _make_pack(8)
_pack2 = _make_pack(2)

_mesh = plsc.VectorSubcoreMesh(
    core_axis_name="c", subcore_axis_name="s", num_cores=NC, num_subcores=NS
)


def _make_sc(nf):
    @functools.partial(
        pl.kernel,
        out_type=jax.ShapeDtypeStruct((nf * D, B), jnp.float32),
        mesh=_mesh,
        scratch_types=[
            pltpu.VMEM((nf * BPW,), jnp.int32),
            pltpu.VMEM((R, 128), jnp.float32),
            pltpu.VMEM((R, 128), jnp.float32),
            pltpu.VMEM((nf * D, R), jnp.float32),
            pltpu.SemaphoreType.DMA,
            pltpu.SemaphoreType.DMA,
            pltpu.SemaphoreType.DMA,
            pltpu.SemaphoreType.DMA,
        ],
        compiler_params=pltpu.CompilerParams(needs_layout_passes=False),
    )
    def sc_gather(*refs):
        feats = refs[:nf]
        group = refs[nf]
        out = refs[nf + 1]
        (idx_v, buf0, buf1, rows_v,
         sem_i, sem_g0, sem_g1, sem_w) = refs[nf + 2:]
        bufs = (buf0, buf1)
        sems = (sem_g0, sem_g1)

        wid = lax.axis_index("s") * NC + lax.axis_index("c")
        base = wid * BPW

        lanes = lax.iota(jnp.int32, L)

        copies = [
            pltpu.async_copy(
                feats[f].at[pl.ds(base, BPW)],
                idx_v.at[pl.ds(f * BPW, BPW)],
                sem_i,
            )
            for f in range(nf)
        ]
        for cp in copies:
            cp.wait()

        def do_chunk(c, carry):
            cbase = base + c * R

            def gather(f):
                return pltpu.async_copy(
                    group.at[idx_v.at[pl.ds(f * BPW + c * R, R)]],
                    bufs[f % 2],
                    sems[f % 2],
                )

            pending = gather(0)
            for f in range(nf):
                nxt = gather(f + 1) if f + 1 < nf else None
                pending.wait()
                buf = bufs[f % 2]
                lane0 = f * D

                def select(blk, carry2):
                    rows = blk * L + lanes
                    for d in range(D):
                        col = jnp.full((L,), lane0 + d, jnp.int32)
                        vals = plsc.load_gather(buf, [rows, col])
                        rows_v[f * D + d, pl.ds(blk * L, L)] = vals
                    return carry2

                lax.fori_loop(0, R // L, select, 0)
                pending = nxt

            pltpu.async_copy(rows_v, out.at[:, pl.ds(cbase, R)], sem_w).wait()
            return carry

        lax.fori_loop(0, NCHUNK, do_chunk, 0)

    return sc_gather


_sc8 = _make_sc(8)
_sc2 = _make_sc(2)


def kernel(feat_0, feat_1, feat_2, feat_3, feat_4, feat_5, feat_6, feat_7, feat_8, feat_9, feat_10, feat_11, feat_12, feat_13, feat_14, feat_15, feat_16, feat_17, feat_18, feat_19, feat_20, feat_21, feat_22, feat_23, feat_24, feat_25, table_0, table_1, table_2, table_3, table_4, table_5, table_6, table_7, table_8, table_9, table_10, table_11, table_12, table_13, table_14, table_15, table_16, table_17, table_18, table_19, table_20, table_21, table_22, table_23, table_24, table_25):
    args = locals()
    feats = [args[f"feat_{i}"] for i in range(NUM_FIELDS)]
    tablesT = [
        jnp.swapaxes(args[f"table_{i}"], 0, 1) for i in range(NUM_FIELDS)
    ]
    bands = []
    f0 = 0
    for nf in GROUP_SIZES:
        pack = _pack8 if nf == 8 else _pack2
        sc = _sc8 if nf == 8 else _sc2
        grp = pack(*tablesT[f0:f0 + nf])
        bands.append(sc(*feats[f0:f0 + nf], grp))
        f0 += nf
    return jnp.swapaxes(jnp.concatenate(bands, axis=0), 0, 1)


# speedup vs baseline: 1.0130x; 1.0130x over previous
import functools

import jax
import jax.numpy as jnp
from jax import lax
from jax.experimental import pallas as pl
from jax.experimental.pallas import tpu as pltpu
from jax.experimental.pallas import tpu_sc as plsc

NUM_FIELDS = 26
VOCAB = 100000
D = 16
B = 16384
OUT_W = NUM_FIELDS * D
GRP = 128 // D
NGRP = -(-NUM_FIELDS // GRP)

NC = 2
NS = 16
NW = NC * NS
BPW = B // NW
R = 128
NCHUNK = BPW // R
L = 16

TBLK = 5120
NSTEP = -(-VOCAB // TBLK)


def _pack_body(*refs):
    t_refs = refs[:NUM_FIELDS]
    o_refs = refs[NUM_FIELDS:]
    for k in range(NGRP):
        fields = range(k * GRP, min((k + 1) * GRP, NUM_FIELDS))
        parts = [t_refs[f][...] for f in fields]
        pad = GRP - len(parts)
        if pad:
            parts += [jnp.zeros_like(parts[0])] * pad
        xk = jnp.concatenate(parts, axis=0)
        o_refs[k][...] = jnp.transpose(xk, (1, 0))


_pack_all = pl.pallas_call(
    _pack_body,
    grid=(NSTEP,),
    in_specs=[pl.BlockSpec((D, TBLK), lambda i: (0, i))] * NUM_FIELDS,
    out_specs=[pl.BlockSpec((TBLK, 128), lambda i: (i, 0))] * NGRP,
    out_shape=[jax.ShapeDtypeStruct((VOCAB, 128), jnp.float32)] * NGRP,
)

_mesh = plsc.VectorSubcoreMesh(
    core_axis_name="c", subcore_axis_name="s", num_cores=NC, num_subcores=NS
)

NIDX = NUM_FIELDS * BPW


@functools.partial(
    pl.kernel,
    out_type=jax.ShapeDtypeStruct((OUT_W, B), jnp.float32),
    mesh=_mesh,
    scratch_types=[
        pltpu.VMEM((NIDX,), jnp.int32),
        pltpu.VMEM((R, 128), jnp.float32),
        pltpu.VMEM((R, 128), jnp.float32),
        pltpu.VMEM((R, 128), jnp.float32),
        pltpu.VMEM((OUT_W, R), jnp.float32),
        pltpu.SemaphoreType.DMA,
        pltpu.SemaphoreType.DMA,
        pltpu.SemaphoreType.DMA,
        pltpu.SemaphoreType.DMA,
        pltpu.SemaphoreType.DMA,
    ],
    compiler_params=pltpu.CompilerParams(needs_layout_passes=False),
)
def _embed_cat(*refs):
    feats = refs[:NUM_FIELDS]
    groups = refs[NUM_FIELDS:NUM_FIELDS + NGRP]
    out = refs[NUM_FIELDS + NGRP]
    (idx_v, buf0, buf1, buf2, rows_v,
     sem_i, sem_g0, sem_g1, sem_g2, sem_w) = refs[NUM_FIELDS + NGRP + 1:]
    bufs = (buf0, buf1, buf2)
    sems = (sem_g0, sem_g1, sem_g2)
    NBUF = 3

    wid = lax.axis_index("s") * NC + lax.axis_index("c")
    base = wid * BPW

    lanes = lax.iota(jnp.int32, L)

    copies = [
        pltpu.async_copy(
            feats[f].at[pl.ds(base, BPW)], idx_v.at[pl.ds(f * BPW, BPW)], sem_i
        )
        for f in range(NUM_FIELDS)
    ]
    for cp in copies:
        cp.wait()

    def do_chunk(c, carry):
        cbase = base + c * R

        def gather(f):
            return pltpu.async_copy(
                groups[f // GRP].at[idx_v.at[pl.ds(f * BPW + c * R, R)]],
                bufs[f % NBUF],
                sems[f % NBUF],
            )

        pendings = [gather(f) for f in range(NBUF - 1)]
        for f in range(NUM_FIELDS):
            if f + NBUF - 1 < NUM_FIELDS:
                pendings.append(gather(f + NBUF - 1))
            pendings.pop(0).wait()
            buf = bufs[f % NBUF]
            lane0 = (f % GRP) * D

            def select(blk, carry2):
                rows = blk * L + lanes
                for d in range(D):
                    col = jnp.full((L,), lane0 + d, jnp.int32)
                    vals = plsc.load_gather(buf, [rows, col])
                    rows_v[f * D + d, pl.ds(blk * L, L)] = vals
                return carry2

            lax.fori_loop(0, R // L, select, 0)

        pltpu.async_copy(rows_v, out.at[:, pl.ds(cbase, R)], sem_w).wait()
        return carry

    lax.fori_loop(0, NCHUNK, do_chunk, 0)


def kernel(feat_0, feat_1, feat_2, feat_3, feat_4, feat_5, feat_6, feat_7, feat_8, feat_9, feat_10, feat_11, feat_12, feat_13, feat_14, feat_15, feat_16, feat_17, feat_18, feat_19, feat_20, feat_21, feat_22, feat_23, feat_24, feat_25, table_0, table_1, table_2, table_3, table_4, table_5, table_6, table_7, table_8, table_9, table_10, table_11, table_12, table_13, table_14, table_15, table_16, table_17, table_18, table_19, table_20, table_21, table_22, table_23, table_24, table_25):
    args = locals()
    feats = [args[f"feat_{i}"] for i in range(NUM_FIELDS)]
    groups = _pack_all(
        *[jnp.swapaxes(args[f"table_{i}"], 0, 1) for i in range(NUM_FIELDS)]
    )
    return jnp.swapaxes(_embed_cat(*feats, *groups), 0, 1)


# speedup vs baseline: 1.0235x; 1.0103x over previous
import functools

import jax
import jax.numpy as jnp
from jax import lax
from jax.experimental import pallas as pl
from jax.experimental.pallas import tpu as pltpu
from jax.experimental.pallas import tpu_sc as plsc

NUM_FIELDS = 26
VOCAB = 100000
D = 16
B = 16384
OUT_W = NUM_FIELDS * D
GRP = 128 // D
NGRP = -(-NUM_FIELDS // GRP)

NC = 2
NS = 16
NW = NC * NS
BPW = B // NW
R = 128
NCHUNK = BPW // R
L = 16

TBLK = 5120
NSTEP = -(-VOCAB // TBLK)


def _pack_body(*refs):
    t_refs = refs[:NUM_FIELDS]
    o_refs = refs[NUM_FIELDS:]
    for k in range(NGRP):
        fields = range(k * GRP, min((k + 1) * GRP, NUM_FIELDS))
        parts = [t_refs[f][...] for f in fields]
        pad = GRP - len(parts)
        if pad:
            parts += [jnp.zeros_like(parts[0])] * pad
        xk = jnp.concatenate(parts, axis=0)
        o_refs[k][...] = jnp.transpose(xk, (1, 0))


_pack_all = pl.pallas_call(
    _pack_body,
    grid=(NSTEP,),
    in_specs=[pl.BlockSpec((D, TBLK), lambda i: (0, i))] * NUM_FIELDS,
    out_specs=[pl.BlockSpec((TBLK, 128), lambda i: (i, 0))] * NGRP,
    out_shape=[jax.ShapeDtypeStruct((VOCAB, 128), jnp.float32)] * NGRP,
)

_mesh = plsc.VectorSubcoreMesh(
    core_axis_name="c", subcore_axis_name="s", num_cores=NC, num_subcores=NS
)

NIDX = NUM_FIELDS * BPW


@functools.partial(
    pl.kernel,
    out_type=jax.ShapeDtypeStruct((OUT_W, B), jnp.float32),
    mesh=_mesh,
    scratch_types=[
        pltpu.VMEM((NIDX,), jnp.int32),
        pltpu.VMEM((R, 128), jnp.float32),
        pltpu.VMEM((R, 128), jnp.float32),
        pltpu.VMEM((OUT_W, R), jnp.float32),
        pltpu.SemaphoreType.DMA,
        pltpu.SemaphoreType.DMA,
        pltpu.SemaphoreType.DMA,
        pltpu.SemaphoreType.DMA,
    ],
    compiler_params=pltpu.CompilerParams(needs_layout_passes=False),
)
def _embed_cat(*refs):
    feats = refs[:NUM_FIELDS]
    groups = refs[NUM_FIELDS:NUM_FIELDS + NGRP]
    out = refs[NUM_FIELDS + NGRP]
    (idx_v, buf0, buf1, rows_v,
     sem_i, sem_g0, sem_g1, sem_w) = refs[NUM_FIELDS + NGRP + 1:]
    bufs = (buf0, buf1)
    sems = (sem_g0, sem_g1)

    wid = lax.axis_index("s") * NC + lax.axis_index("c")
    base = wid * BPW

    lanes = lax.iota(jnp.int32, L)

    copies = [
        pltpu.async_copy(
            feats[f].at[pl.ds(base, BPW)], idx_v.at[pl.ds(f * BPW, BPW)], sem_i
        )
        for f in range(NUM_FIELDS)
    ]
    for cp in copies:
        cp.wait()

    def do_chunk(c, carry):
        cbase = base + c * R

        def gather(f):
            return pltpu.async_copy(
                groups[f // GRP].at[idx_v.at[pl.ds(f * BPW + c * R, R)]],
                bufs[f % 2],
                sems[f % 2],
            )

        pending = gather(0)
        for f in range(NUM_FIELDS):
            nxt = gather(f + 1) if f + 1 < NUM_FIELDS else None
            pending.wait()
            buf = bufs[f % 2]
            lane0 = (f % GRP) * D

            def select(blk, carry2):
                rows = blk * L + lanes
                for d in range(D):
                    col = jnp.full((L,), lane0 + d, jnp.int32)
                    vals = plsc.load_gather(buf, [rows, col])
                    rows_v[f * D + d, pl.ds(blk * L, L)] = vals
                return carry2

            lax.fori_loop(0, R // L, select, 0)
            pending = nxt

        pltpu.async_copy(rows_v, out.at[:, pl.ds(cbase, R)], sem_w).wait()
        return carry

    lax.fori_loop(0, NCHUNK, do_chunk, 0)


def kernel(feat_0, feat_1, feat_2, feat_3, feat_4, feat_5, feat_6, feat_7, feat_8, feat_9, feat_10, feat_11, feat_12, feat_13, feat_14, feat_15, feat_16, feat_17, feat_18, feat_19, feat_20, feat_21, feat_22, feat_23, feat_24, feat_25, table_0, table_1, table_2, table_3, table_4, table_5, table_6, table_7, table_8, table_9, table_10, table_11, table_12, table_13, table_14, table_15, table_16, table_17, table_18, table_19, table_20, table_21, table_22, table_23, table_24, table_25):
    args = locals()
    feats = [args[f"feat_{i}"] for i in range(NUM_FIELDS)]
    groups = _pack_all(
        *[jnp.swapaxes(args[f"table_{i}"], 0, 1) for i in range(NUM_FIELDS)]
    )
    return jnp.swapaxes(_embed_cat(*feats, *groups), 0, 1)
